# Initial kernel scaffold; baseline (speedup 1.0000x reference)
import jax
import jax.numpy as jnp
from jax.experimental import pallas as pl


def kernel(x, thresholds, ordinals, weights):
    raise NotImplementedError("write your pallas kernel here")


# speedup vs baseline: 6324.8784x; 6324.8784x over previous
import functools

import jax
import jax.numpy as jnp
from jax import lax
from jax.experimental import pallas as pl
from jax.experimental.pallas import tpu as pltpu
from jax.experimental.pallas import tpu_sc as plsc

CIN = 32
COUT = 32
DEPTH = 4
NNODES = 15
NLEAF = 16
PAD = 2
BATCH = 8
LENGTH = 2048

LANES = 16
LSPLIT = 4
LT = LENGTH // LSPLIT
ROW = LT + 128
U = 2


def _hinge_body(xp, thf, orf, wf, out, xtile, thv, orv, wv, accv):
    c = lax.axis_index("c")
    s = lax.axis_index("s")
    wid = s * 2 + c
    b = wid // LSPLIT
    l0 = (wid % LSPLIT) * LT

    pltpu.sync_copy(thf, thv)
    pltpu.sync_copy(orf, orv)
    pltpu.sync_copy(wf, wv)
    pltpu.sync_copy(xp.at[b, :, pl.ds(l0, ROW)], xtile)

    lane = lax.broadcasted_iota(jnp.int32, (LANES,), 0)

    def cout_body(co, _):
        tb0 = co * (CIN * NNODES)
        wb0 = co * (CIN * NLEAF) - NNODES

        def lbg_body(g, _):
            lstart = g * (LANES * U)
            lpos = [lane + (lstart + u * LANES) for u in range(U)]

            def cin_body(ci, accs):
                tb = jnp.full((LANES,), tb0 + ci * NNODES, jnp.int32)
                wb = jnp.full((LANES,), wb0 + ci * NLEAF, jnp.int32)
                cv = jnp.full((LANES,), ci, jnp.int32)
                res = []
                for u in range(U):
                    node = jnp.zeros((LANES,), jnp.int32)
                    minm = None
                    for _lev in range(DEPTH):
                        idx = node + tb
                        t = plsc.load_gather(thv, [idx])
                        o = plsc.load_gather(orv, [idx])
                        v = plsc.load_gather(xtile, [cv, lpos[u] + o])
                        m = v - t
                        am = jnp.abs(m)
                        minm = am if minm is None else jnp.minimum(minm, am)
                        node = node + node + jnp.where(m > 0.0, 2, 1)
                    wgt = plsc.load_gather(wv, [node + wb])
                    res.append(accs[u] + wgt * minm)
                return tuple(res)

            accs = tuple(jnp.zeros((LANES,), jnp.float32) for _ in range(U))
            accs = lax.fori_loop(0, CIN, cin_body, accs)
            for u in range(U):
                accv[co, pl.ds(lstart + u * LANES, LANES)] = accs[u]
            return 0

        lax.fori_loop(0, LT // (LANES * U), lbg_body, 0)
        return 0

    lax.fori_loop(0, COUT, cout_body, 0)
    pltpu.sync_copy(accv, out.at[b, :, pl.ds(l0, LT)])


@jax.jit
def _hinge(xp, thf, orf, wf):
    mesh = plsc.VectorSubcoreMesh(core_axis_name="c", subcore_axis_name="s")
    f = functools.partial(
        pl.kernel,
        mesh=mesh,
        compiler_params=pltpu.CompilerParams(needs_layout_passes=False),
        out_type=jax.ShapeDtypeStruct((BATCH, COUT, LENGTH), jnp.float32),
        scratch_types=[
            pltpu.VMEM((CIN, ROW), jnp.float32),
            pltpu.VMEM((COUT * CIN * NNODES,), jnp.float32),
            pltpu.VMEM((COUT * CIN * NNODES,), jnp.int32),
            pltpu.VMEM((COUT * CIN * NLEAF,), jnp.float32),
            pltpu.VMEM((COUT, LT), jnp.float32),
        ],
    )(_hinge_body)
    return f(xp, thf, orf, wf)


def kernel(x, thresholds, ordinals, weights):
    xp = jnp.pad(x, ((0, 0), (0, 0), (PAD, 126)))
    return _hinge(xp, thresholds.reshape(-1), ordinals.reshape(-1),
                  weights.reshape(-1))
